# baseline (speedup 1.0000x reference)
import functools

import jax
import jax.numpy as jnp
from jax import lax
from jax.experimental import pallas as pl
from jax.experimental.pallas import tpu as pltpu
from jax.experimental.pallas import tpu_sc as plsc

N = 10000
NPAD = 10240
E = 320000
D = 128
G = 64
NC = 2
NS = 16
NW = NC * NS
L = 16

EPW0 = 9984
TAIL_BASE = NW * EPW0
TAIL_PER_TILE = 128

CH = NPAD // NS

TB = 1024
NB = NPAD // TB


def _tc1_body(x_ref, wr_ref, wo_ref, b_ref, srel_ref, sroot_ref):
    w2t = jnp.concatenate([wr_ref[...].reshape(1, D),
                           wo_ref[...].reshape(1, D)], axis=0)
    s2t = lax.dot_general(w2t, x_ref[...], (((1,), (1,)), ((), ())),
                          preferred_element_type=jnp.float32,
                          precision=lax.Precision.HIGHEST)
    srel_ref[...] = s2t[0, :]
    sroot_ref[...] = s2t[1, :] + b_ref[0]


def _tc1(x, w_rel, w_root, b_rel):
    return pl.pallas_call(
        _tc1_body,
        grid=(NB,),
        in_specs=[
            pl.BlockSpec((TB, D), lambda i: (i, 0)),
            pl.BlockSpec((D, 1), lambda i: (0, 0)),
            pl.BlockSpec((D, 1), lambda i: (0, 0)),
            pl.BlockSpec((1,), lambda i: (0,)),
        ],
        out_specs=[
            pl.BlockSpec((TB,), lambda i: (i,)),
            pl.BlockSpec((TB,), lambda i: (i,)),
        ],
        out_shape=[
            jax.ShapeDtypeStruct((NPAD,), jnp.float32),
            jax.ShapeDtypeStruct((NPAD,), jnp.float32),
        ],
    )(x, w_rel, w_root, b_rel)


def _sc_scatter_body(edge_hbm, srel_hbm, out_hbm, eslab_v, etail_v, chunk_v,
                     srel_v, acc_v, srel_sh, sem_e, sem_t, sem_s):
    cid = lax.axis_index("c")
    sid = lax.axis_index("s")
    wid = sid * NC + cid

    cp_e = pltpu.async_copy(edge_hbm.at[:, pl.ds(wid * EPW0, EPW0)],
                            eslab_v, sem_e)

    @pl.when(wid < 4)
    def _():
        pltpu.async_copy(
            edge_hbm.at[:, pl.ds(TAIL_BASE + wid * TAIL_PER_TILE,
                                 TAIL_PER_TILE)], etail_v, sem_t)

    pltpu.sync_copy(srel_hbm.at[pl.ds(sid * CH, CH)], chunk_v)
    pltpu.sync_copy(chunk_v, srel_sh.at[pl.ds(sid * CH, CH)])

    def _zero(i, carry):
        acc_v[pl.ds(i * L, L)] = jnp.zeros((L,), jnp.float32)
        return carry

    lax.fori_loop(0, N // L, _zero, 0)

    plsc.subcore_barrier()
    cp_s = pltpu.async_copy(srel_sh, srel_v, sem_s)
    cp_s.wait()
    cp_e.wait()

    def _edges(i, carry):
        s = eslab_v[0, pl.ds(i * L, L)]
        d = eslab_v[1, pl.ds(i * L, L)]
        v = plsc.load_gather(srel_v, [s])
        plsc.addupdate_scatter(acc_v, [d], v)
        return carry

    lax.fori_loop(0, EPW0 // L, _edges, 0, unroll=4)

    @pl.when(wid < 4)
    def _():
        pltpu.make_async_copy(
            edge_hbm.at[:, pl.ds(TAIL_BASE, TAIL_PER_TILE)],
            etail_v, sem_t).wait()

        def _tail(i, carry):
            s = etail_v[0, pl.ds(i * L, L)]
            d = etail_v[1, pl.ds(i * L, L)]
            v = plsc.load_gather(srel_v, [s])
            plsc.addupdate_scatter(acc_v, [d], v)
            return carry

        lax.fori_loop(0, TAIL_PER_TILE // L, _tail, 0, unroll=4)

    pltpu.sync_copy(acc_v, out_hbm.at[wid])


@functools.cache
def _sc_scatter():
    mesh = plsc.VectorSubcoreMesh(core_axis_name="c", subcore_axis_name="s",
                                  num_cores=NC, num_subcores=NS)
    return pl.kernel(
        _sc_scatter_body,
        out_type=jax.ShapeDtypeStruct((NW, N), jnp.float32),
        mesh=mesh,
        compiler_params=pltpu.CompilerParams(needs_layout_passes=False),
        scratch_types=[
            pltpu.VMEM((2, EPW0), jnp.int32),
            pltpu.VMEM((2, TAIL_PER_TILE), jnp.int32),
            pltpu.VMEM((CH,), jnp.float32),
            pltpu.VMEM((NPAD,), jnp.float32),
            pltpu.VMEM((N,), jnp.float32),
            pltpu.VMEM_SHARED((NPAD,), jnp.float32),
            pltpu.SemaphoreType.DMA,
            pltpu.SemaphoreType.DMA,
            pltpu.SemaphoreType.DMA,
        ],
    )


def _tc2a_body(part_ref, sroot_ref, batch_ref, scores_ref, batchp_ref):
    xconv = jnp.sum(part_ref[...], axis=0) + sroot_ref[pl.ds(0, N)]
    b = batch_ref[...]
    seg = lax.broadcasted_iota(jnp.int32, (G, N), 0)
    mask = seg == b[None, :]
    neg = jnp.float32(-jnp.inf)
    xb = jnp.where(mask, xconv[None, :], neg)
    m = jnp.max(xb, axis=1)
    m = jnp.where(jnp.isfinite(m), m, 0.0)
    t = jnp.where(mask, xconv[None, :] - m[:, None], neg)
    e = jnp.exp(t)
    denom = jnp.sum(e, axis=1)
    m_sel = jnp.sum(jnp.where(mask, m[:, None], 0.0), axis=0)
    d_sel = jnp.sum(jnp.where(mask, denom[:, None], 0.0), axis=0)
    scores_ref[pl.ds(0, N)] = jnp.exp(xconv - m_sel) / (d_sel + 1e-16)
    scores_ref[pl.ds(N, NPAD - N)] = jnp.zeros((NPAD - N,), jnp.float32)
    batchp_ref[pl.ds(0, N)] = b
    batchp_ref[pl.ds(N, NPAD - N)] = jnp.full((NPAD - N,), G, jnp.int32)


def _tc2a(part, sroot, batch):
    return pl.pallas_call(
        _tc2a_body,
        out_shape=[
            jax.ShapeDtypeStruct((NPAD,), jnp.float32),
            jax.ShapeDtypeStruct((NPAD,), jnp.int32),
        ],
    )(part, sroot, batch)


def _tc2b_body(x_ref, sc_ref, batchp_ref, out_ref):
    i = pl.program_id(0)

    @pl.when(i == 0)
    def _():
        out_ref[...] = jnp.zeros_like(out_ref)

    row = lax.broadcasted_iota(jnp.int32, (TB, D), 0) + i * TB
    xb = jnp.where(row < N, x_ref[...], 0.0)
    seg = lax.broadcasted_iota(jnp.int32, (G, TB), 0)
    s = jnp.where(seg == batchp_ref[...][None, :], sc_ref[...][None, :], 0.0)
    out_ref[...] += jnp.dot(s, xb, preferred_element_type=jnp.float32)


def _tc2b(x, scores, batchp):
    return pl.pallas_call(
        _tc2b_body,
        grid=(NB,),
        in_specs=[
            pl.BlockSpec((TB, D), lambda i: (i, 0)),
            pl.BlockSpec((TB,), lambda i: (i,)),
            pl.BlockSpec((TB,), lambda i: (i,)),
        ],
        out_specs=pl.BlockSpec((G, D), lambda i: (0, 0)),
        out_shape=jax.ShapeDtypeStruct((G, D), jnp.float32),
    )(x, scores, batchp)


def kernel(x, W_rel, b_rel, W_root, edge_index, batch):
    srel, sroot = _tc1(x, W_rel, W_root, b_rel)
    part = _sc_scatter()(edge_index, srel)
    scores, batchp = _tc2a(part, sroot, batch)
    return _tc2b(x, scores, batchp)


# speedup vs baseline: 1.3327x; 1.3327x over previous
import functools

import jax
import jax.numpy as jnp
from jax import lax
from jax.experimental import pallas as pl
from jax.experimental.pallas import tpu as pltpu
from jax.experimental.pallas import tpu_sc as plsc

N = 10000
E = 320000
D = 128
G = 64
NC = 2
NS = 16
NW = NC * NS
L = 16

EPW0 = 9984
TAIL_BASE = NW * EPW0
TAIL_PER_TILE = 128


NPAD = 10240
TB = 1024
NB = NPAD // TB


def _tc1_body(x_ref, w2t_ref, b_ref, srel_ref, sroot_ref):
    s2t = lax.dot_general(w2t_ref[...], x_ref[...], (((1,), (1,)), ((), ())),
                          preferred_element_type=jnp.float32,
                          precision=lax.Precision.HIGHEST)
    i = pl.program_id(0)
    srel_ref[pl.ds(i * TB, TB)] = s2t[0, :]
    sroot_ref[pl.ds(i * TB, TB)] = s2t[1, :] + b_ref[0, 0]


def _tc1(x, w2t, b_rel):
    return pl.pallas_call(
        _tc1_body,
        grid=(NB,),
        in_specs=[
            pl.BlockSpec((TB, D), lambda i: (i, 0)),
            pl.BlockSpec((2, D), lambda i: (0, 0)),
            pl.BlockSpec((1, 1), lambda i: (0, 0)),
        ],
        out_specs=[
            pl.BlockSpec((NPAD,), lambda i: (0,)),
            pl.BlockSpec((NPAD,), lambda i: (0,)),
        ],
        out_shape=[
            jax.ShapeDtypeStruct((NPAD,), jnp.float32),
            jax.ShapeDtypeStruct((NPAD,), jnp.float32),
        ],
    )(x, w2t, b_rel)


def _sc_scatter_body(edge_hbm, srel_hbm, out_hbm, eslab_v, etail_v, srel_v,
                     acc_v, sem_e, sem_t, sem_s):
    wid = lax.axis_index("s") * NC + lax.axis_index("c")

    cp_e = pltpu.async_copy(edge_hbm.at[:, pl.ds(wid * EPW0, EPW0)],
                            eslab_v, sem_e)
    cp_s = pltpu.async_copy(srel_hbm, srel_v, sem_s)

    @pl.when(wid < 4)
    def _():
        pltpu.async_copy(
            edge_hbm.at[:, pl.ds(TAIL_BASE + wid * TAIL_PER_TILE,
                                 TAIL_PER_TILE)], etail_v, sem_t)

    @plsc.parallel_loop(0, N // L, unroll=8)
    def _(i):
        acc_v[pl.ds(i * L, L)] = jnp.zeros((L,), jnp.float32)

    cp_s.wait()
    cp_e.wait()

    @plsc.parallel_loop(0, EPW0 // L, unroll=4)
    def _(i):
        s = eslab_v[0, pl.ds(i * L, L)]
        d = eslab_v[1, pl.ds(i * L, L)]
        v = plsc.load_gather(srel_v, [s])
        plsc.addupdate_scatter(acc_v, [d], v)

    @pl.when(wid < 4)
    def _():
        pltpu.make_async_copy(
            edge_hbm.at[:, pl.ds(TAIL_BASE, TAIL_PER_TILE)],
            etail_v, sem_t).wait()

        def _tail(i, carry):
            s = etail_v[0, pl.ds(i * L, L)]
            d = etail_v[1, pl.ds(i * L, L)]
            v = plsc.load_gather(srel_v, [s])
            plsc.addupdate_scatter(acc_v, [d], v)
            return carry

        lax.fori_loop(0, TAIL_PER_TILE // L, _tail, 0, unroll=4)

    pltpu.sync_copy(acc_v, out_hbm.at[wid])


@functools.cache
def _sc_scatter():
    mesh = plsc.VectorSubcoreMesh(core_axis_name="c", subcore_axis_name="s",
                                  num_cores=NC, num_subcores=NS)
    return pl.kernel(
        _sc_scatter_body,
        out_type=jax.ShapeDtypeStruct((NW, N), jnp.float32),
        mesh=mesh,
        compiler_params=pltpu.CompilerParams(needs_layout_passes=False),
        scratch_types=[
            pltpu.VMEM((2, EPW0), jnp.int32),
            pltpu.VMEM((2, TAIL_PER_TILE), jnp.int32),
            pltpu.VMEM((NPAD,), jnp.float32),
            pltpu.VMEM((N,), jnp.float32),
            pltpu.SemaphoreType.DMA,
            pltpu.SemaphoreType.DMA,
            pltpu.SemaphoreType.DMA,
        ],
    )


def _tc2_body(x_ref, part_ref, sroot_ref, batch_ref, out_ref):
    xconv = jnp.sum(part_ref[...], axis=0) + sroot_ref[pl.ds(0, N)]
    b = batch_ref[...]
    seg = lax.broadcasted_iota(jnp.int32, (G, N), 0)
    mask = seg == b[None, :]
    neg = jnp.float32(-jnp.inf)
    xb = jnp.where(mask, xconv[None, :], neg)
    m = jnp.max(xb, axis=1)
    m = jnp.where(jnp.isfinite(m), m, 0.0)
    t = jnp.where(mask, xconv[None, :] - m[:, None], neg)
    e = jnp.exp(t)
    denom = jnp.sum(e, axis=1)
    scores = e / (denom[:, None] + 1e-16)
    out_ref[...] = jnp.dot(scores, x_ref[...],
                           preferred_element_type=jnp.float32)


def _tc2(x, part, sroot, batch):
    return pl.pallas_call(
        _tc2_body,
        out_shape=jax.ShapeDtypeStruct((G, D), jnp.float32),
    )(x, part, sroot, batch)


def kernel(x, W_rel, b_rel, W_root, edge_index, batch):
    w2t = jnp.concatenate([W_rel.reshape(1, D), W_root.reshape(1, D)], axis=0)
    srel, sroot = _tc1(x, w2t, b_rel.reshape(1, 1))
    part = _sc_scatter()(edge_index, srel)
    return _tc2(x, part, sroot, batch)


# speedup vs baseline: 1.4075x; 1.0561x over previous
import functools

import jax
import jax.numpy as jnp
from jax import lax
from jax.experimental import pallas as pl
from jax.experimental.pallas import tpu as pltpu
from jax.experimental.pallas import tpu_sc as plsc

N = 10000
E = 320000
D = 128
G = 64
NC = 2
NS = 16
NW = NC * NS
L = 16

EPW0 = 9984
TAIL_BASE = NW * EPW0
TAIL_PER_TILE = 128


def _tc1_body(x_ref, w2t_ref, b_ref, srel_ref, sroot_ref):
    s2t = lax.dot_general(w2t_ref[...], x_ref[...], (((1,), (1,)), ((), ())),
                          preferred_element_type=jnp.float32,
                          precision=lax.Precision.HIGHEST)
    srel_ref[...] = s2t[0, :]
    sroot_ref[...] = s2t[1, :] + b_ref[0, 0]


def _tc1(x, w2t, b_rel):
    return pl.pallas_call(
        _tc1_body,
        out_shape=[
            jax.ShapeDtypeStruct((N,), jnp.float32),
            jax.ShapeDtypeStruct((N,), jnp.float32),
        ],
    )(x, w2t, b_rel)


def _sc_scatter_body(edge_hbm, srel_hbm, out_hbm, eslab_v, etail_v, srel_v,
                     acc_v, sem_e, sem_t, sem_s):
    wid = lax.axis_index("s") * NC + lax.axis_index("c")

    cp_e = pltpu.async_copy(edge_hbm.at[:, pl.ds(wid * EPW0, EPW0)],
                            eslab_v, sem_e)
    cp_s = pltpu.async_copy(srel_hbm, srel_v, sem_s)

    @pl.when(wid < 4)
    def _():
        pltpu.async_copy(
            edge_hbm.at[:, pl.ds(TAIL_BASE + wid * TAIL_PER_TILE,
                                 TAIL_PER_TILE)], etail_v, sem_t)

    @plsc.parallel_loop(0, N // L, unroll=8)
    def _(i):
        acc_v[pl.ds(i * L, L)] = jnp.zeros((L,), jnp.float32)

    cp_s.wait()
    cp_e.wait()

    @plsc.parallel_loop(0, EPW0 // L, unroll=8)
    def _(i):
        s = eslab_v[0, pl.ds(i * L, L)]
        d = eslab_v[1, pl.ds(i * L, L)]
        v = plsc.load_gather(srel_v, [s])
        plsc.addupdate_scatter(acc_v, [d], v)

    @pl.when(wid < 4)
    def _():
        pltpu.make_async_copy(
            edge_hbm.at[:, pl.ds(TAIL_BASE, TAIL_PER_TILE)],
            etail_v, sem_t).wait()

        def _tail(i, carry):
            s = etail_v[0, pl.ds(i * L, L)]
            d = etail_v[1, pl.ds(i * L, L)]
            v = plsc.load_gather(srel_v, [s])
            plsc.addupdate_scatter(acc_v, [d], v)
            return carry

        lax.fori_loop(0, TAIL_PER_TILE // L, _tail, 0, unroll=4)

    pltpu.sync_copy(acc_v, out_hbm.at[wid])


@functools.cache
def _sc_scatter():
    mesh = plsc.VectorSubcoreMesh(core_axis_name="c", subcore_axis_name="s",
                                  num_cores=NC, num_subcores=NS)
    return pl.kernel(
        _sc_scatter_body,
        out_type=jax.ShapeDtypeStruct((NW, N), jnp.float32),
        mesh=mesh,
        compiler_params=pltpu.CompilerParams(needs_layout_passes=False),
        scratch_types=[
            pltpu.VMEM((2, EPW0), jnp.int32),
            pltpu.VMEM((2, TAIL_PER_TILE), jnp.int32),
            pltpu.VMEM((N,), jnp.float32),
            pltpu.VMEM((N,), jnp.float32),
            pltpu.SemaphoreType.DMA,
            pltpu.SemaphoreType.DMA,
            pltpu.SemaphoreType.DMA,
        ],
    )


def _tc2_body(x_ref, part_ref, sroot_ref, batch_ref, out_ref):
    xconv = jnp.sum(part_ref[...], axis=0) + sroot_ref[...]
    b = batch_ref[...]
    seg = lax.broadcasted_iota(jnp.int32, (G, N), 0)
    mask = seg == b[None, :]
    neg = jnp.float32(-jnp.inf)
    xb = jnp.where(mask, xconv[None, :], neg)
    m = jnp.max(xb, axis=1)
    m = jnp.where(jnp.isfinite(m), m, 0.0)
    t = jnp.where(mask, xconv[None, :] - m[:, None], neg)
    e = jnp.exp(t)
    denom = jnp.sum(e, axis=1)
    scores = e / (denom[:, None] + 1e-16)
    out_ref[...] = jnp.dot(scores, x_ref[...],
                           preferred_element_type=jnp.float32)


def _tc2(x, part, sroot, batch):
    return pl.pallas_call(
        _tc2_body,
        out_shape=jax.ShapeDtypeStruct((G, D), jnp.float32),
    )(x, part, sroot, batch)


def kernel(x, W_rel, b_rel, W_root, edge_index, batch):
    w2t = jnp.concatenate([W_rel.reshape(1, D), W_root.reshape(1, D)], axis=0)
    srel, sroot = _tc1(x, w2t, b_rel.reshape(1, 1))
    part = _sc_scatter()(edge_index, srel)
    return _tc2(x, part, sroot, batch)
